# baseline (speedup 1.0000x reference)
import functools

import jax
import jax.numpy as jnp
from jax import lax
from jax.experimental import pallas as pl
from jax.experimental.pallas import tpu as pltpu
from jax.experimental.pallas import tpu_sc as plsc

ENTITY_N = 1000000
RELATION_N = 1000
HIDDEN = 64
BATCH = 16384
MARGIN = 1.0
LANES = 16
NGRP = HIDDEN // LANES
PAIR_W = 2 * HIDDEN

NC = 2
NS = 16
NW = NC * NS

ROWS = 2 * BATCH
RPW = ROWS // NW
CHUNK = 128
NCHUNK = RPW // CHUNK
OUT_W = 128
OUT_RPW = RPW * LANES // OUT_W
OUT_CPC = CHUNK * LANES // OUT_W


def _sc_body(ent2, rel2, hidx, tidx, ridx, out,
             ih, it, ir, gh0, gt0, gr0, gh1, gt1, gr1,
             hv0, tv0, rv0, hv1, tv1, rv1,
             acc0, acc1, sem0, sem1, osem):
    wid = lax.axis_index("s") * NC + lax.axis_index("c")
    base = wid * RPW
    pltpu.sync_copy(hidx.at[pl.ds(base, RPW)], ih.at[pl.ds(0, RPW)])
    pltpu.sync_copy(tidx.at[pl.ds(base, RPW)], it.at[pl.ds(0, RPW)])
    pltpu.sync_copy(ridx.at[pl.ds(base, RPW)], ir.at[pl.ds(0, RPW)])
    bufs = ((gh0, gt0, gr0, hv0, tv0, rv0, acc0, sem0),
            (gh1, gt1, gr1, hv1, tv1, rv1, acc1, sem1))

    def fire(c, bi):
        gh, gt, gr, hv, tv, rv, _, sem = bufs[bi]
        off = c * CHUNK
        for v in range(CHUNK // LANES):
            sl = pl.ds(v * LANES, LANES)
            osl = pl.ds(off + v * LANES, LANES)
            gh[sl] = ih[osl] >> 1
            gt[sl] = it[osl] >> 1
            gr[sl] = ir[osl] >> 1
        pltpu.async_copy(ent2.at[gh], hv, sem)
        pltpu.async_copy(ent2.at[gt], tv, sem)
        pltpu.async_copy(rel2.at[gr], rv, sem)

    def consume(c, bi):
        _, _, _, hv, tv, rv, acc, sem = bufs[bi]
        off = c * CHUNK
        pltpu.make_async_copy(ent2.at[pl.ds(0, CHUNK)], hv, sem).wait()
        pltpu.make_async_copy(ent2.at[pl.ds(0, CHUNK)], tv, sem).wait()
        pltpu.make_async_copy(ent2.at[pl.ds(0, CHUNK)], rv, sem).wait()

        @plsc.parallel_loop(0, CHUNK, unroll=4)
        def cbody(j):
            vh = ih[pl.ds(off + j, LANES)]
            vt = it[pl.ds(off + j, LANES)]
            vr = ir[pl.ds(off + j, LANES)]
            ph = (vh[0] & 1) * HIDDEN
            pt = (vt[0] & 1) * HIDDEN
            pr = (vr[0] & 1) * HIDDEN
            a = None
            for k in range(NGRP):
                ko = k * LANES
                d = hv[j, pl.ds(ph + ko, LANES)] \
                    - tv[j, pl.ds(pt + ko, LANES)] \
                    + rv[j, pl.ds(pr + ko, LANES)]
                sq = d * d
                a = sq if a is None else a + sq
            acc[j >> 3, pl.ds((j & 7) * LANES, LANES)] = a

        pltpu.async_copy(
            acc, out.at[pl.ds(wid * OUT_RPW + c * OUT_CPC, OUT_CPC)], osem)

    fire(0, 0)
    for c in range(NCHUNK):
        if c + 1 < NCHUNK:
            fire(c + 1, (c + 1) % 2)
        if c >= 2:
            pltpu.make_async_copy(
                bufs[c % 2][6], out.at[pl.ds(0, OUT_CPC)], osem).wait()
        consume(c, c % 2)
    pltpu.make_async_copy(
        bufs[0][6], out.at[pl.ds(0, OUT_CPC)], osem).wait()
    pltpu.make_async_copy(
        bufs[1][6], out.at[pl.ds(0, OUT_CPC)], osem).wait()


_sc_partials = functools.partial(
    pl.kernel,
    out_type=jax.ShapeDtypeStruct((ROWS * LANES // OUT_W, OUT_W), jnp.float32),
    mesh=plsc.VectorSubcoreMesh(core_axis_name="c", subcore_axis_name="s"),
    scratch_types=[
        pltpu.VMEM((RPW + LANES,), jnp.int32),
        pltpu.VMEM((RPW + LANES,), jnp.int32),
        pltpu.VMEM((RPW + LANES,), jnp.int32),
        pltpu.VMEM((CHUNK,), jnp.int32),
        pltpu.VMEM((CHUNK,), jnp.int32),
        pltpu.VMEM((CHUNK,), jnp.int32),
        pltpu.VMEM((CHUNK,), jnp.int32),
        pltpu.VMEM((CHUNK,), jnp.int32),
        pltpu.VMEM((CHUNK,), jnp.int32),
        pltpu.VMEM((CHUNK, PAIR_W), jnp.float32),
        pltpu.VMEM((CHUNK, PAIR_W), jnp.float32),
        pltpu.VMEM((CHUNK, PAIR_W), jnp.float32),
        pltpu.VMEM((CHUNK, PAIR_W), jnp.float32),
        pltpu.VMEM((CHUNK, PAIR_W), jnp.float32),
        pltpu.VMEM((CHUNK, PAIR_W), jnp.float32),
        pltpu.VMEM((OUT_CPC, OUT_W), jnp.float32),
        pltpu.VMEM((OUT_CPC, OUT_W), jnp.float32),
        pltpu.SemaphoreType.DMA,
        pltpu.SemaphoreType.DMA,
        pltpu.SemaphoreType.DMA,
    ],
)(_sc_body)


def _tc_finish(parts_ref, out_ref):
    x = parts_ref[...]
    g = lax.broadcasted_iota(jnp.int32, (OUT_W, OUT_W // LANES), 0)
    h = lax.broadcasted_iota(jnp.int32, (OUT_W, OUT_W // LANES), 1)
    m = (g // LANES == h).astype(jnp.float32)
    s = jax.lax.dot_general(x, m, (((1,), (0,)), ((), ())),
                            preferred_element_type=jnp.float32)
    sc = jnp.sqrt(s)
    half = sc.shape[0] // 2
    val = jnp.maximum(sc[:half] - sc[half:] + MARGIN, 0.0)
    out_ref[0, 0] = jnp.sum(val)


def kernel(pos_h, pos_r, pos_t, neg_h, neg_r, neg_t,
           entity_embeddings, relation_embeddings):
    hidx = jnp.concatenate([pos_h, neg_h]).astype(jnp.int32)
    tidx = jnp.concatenate([pos_t, neg_t]).astype(jnp.int32)
    ridx = jnp.concatenate([pos_r[:, 0], neg_r[:, 0]]).astype(jnp.int32)
    ent2 = entity_embeddings.reshape(ENTITY_N // 2, PAIR_W)
    rel2 = relation_embeddings.reshape(RELATION_N // 2, PAIR_W)
    parts = _sc_partials(ent2, rel2, hidx, tidx, ridx)
    loss = pl.pallas_call(
        _tc_finish,
        out_shape=jax.ShapeDtypeStruct((1, 1), jnp.float32),
        out_specs=pl.BlockSpec(memory_space=pltpu.SMEM),
    )(parts)
    return loss.reshape(())


# speedup vs baseline: 2.4108x; 2.4108x over previous
import functools

import jax
import jax.numpy as jnp
from jax import lax
from jax.experimental import pallas as pl
from jax.experimental.pallas import tpu as pltpu
from jax.experimental.pallas import tpu_sc as plsc

ENTITY_N = 1000000
RELATION_N = 1000
HIDDEN = 64
BATCH = 16384
MARGIN = 1.0
LANES = 16
NGRP = HIDDEN // LANES
SUB = 8

NC = 2
NS = 16
NW = NC * NS

ROWS = 2 * BATCH
RPW = ROWS // NW
CHUNK = 128
NCHUNK = RPW // CHUNK
CSUB = CHUNK // SUB
OUT_W = 128
OUT_RPW = RPW * LANES // OUT_W
OUT_CPC = CHUNK * LANES // OUT_W

ENT_T = ENTITY_N // SUB
REL_T = RELATION_N // SUB


def _sc_body(ent3, rel3, hidx, tidx, ridx, out,
             ih, it, ir, hv0, tv0, rv0, hv1, tv1, rv1,
             acc0, acc1, sem0, sem1, osem):
    wid = lax.axis_index("s") * NC + lax.axis_index("c")
    base = wid * RPW
    pltpu.sync_copy(hidx.at[pl.ds(base, RPW)], ih)
    pltpu.sync_copy(tidx.at[pl.ds(base, RPW)], it)
    pltpu.sync_copy(ridx.at[pl.ds(base, RPW)], ir)
    bufs = ((hv0, tv0, rv0, acc0, sem0), (hv1, tv1, rv1, acc1, sem1))

    def fire(c, bi):
        hv, tv, rv, _, sem = bufs[bi]
        off = c * CHUNK

        @plsc.parallel_loop(0, CHUNK // LANES, unroll=2)
        def fblock(b):
            j0 = b * LANES
            vh = ih[pl.ds(off + j0, LANES)]
            vt = it[pl.ds(off + j0, LANES)]
            vr = ir[pl.ds(off + j0, LANES)]
            for i in range(LANES):
                j = j0 + i
                h = vh[i]
                pltpu.async_copy(ent3.at[h >> 3, h & 7],
                                 hv.at[j >> 3, j & 7], sem)
                t = vt[i]
                pltpu.async_copy(ent3.at[t >> 3, t & 7],
                                 tv.at[j >> 3, j & 7], sem)
                r = vr[i]
                pltpu.async_copy(rel3.at[r >> 3, r & 7],
                                 rv.at[j >> 3, j & 7], sem)

    def consume(c, bi):
        hv, tv, rv, acc, sem = bufs[bi]
        pltpu.make_async_copy(ent3.at[pl.ds(0, CSUB)], hv, sem).wait()
        pltpu.make_async_copy(ent3.at[pl.ds(0, CSUB)], tv, sem).wait()
        pltpu.make_async_copy(rel3.at[pl.ds(0, CSUB)], rv, sem).wait()

        @plsc.parallel_loop(0, CHUNK, unroll=4)
        def cbody(j):
            a = None
            for k in range(NGRP):
                sl = pl.ds(k * LANES, LANES)
                d = hv[j >> 3, j & 7, sl] - tv[j >> 3, j & 7, sl] \
                    + rv[j >> 3, j & 7, sl]
                sq = d * d
                a = sq if a is None else a + sq
            acc[j >> 3, pl.ds((j & 7) * LANES, LANES)] = a
        pltpu.async_copy(
            acc, out.at[pl.ds(wid * OUT_RPW + c * OUT_CPC, OUT_CPC)], osem)

    fire(0, 0)
    for c in range(NCHUNK):
        if c + 1 < NCHUNK:
            fire(c + 1, (c + 1) % 2)
        if c >= 2:
            pltpu.make_async_copy(
                bufs[c % 2][3], out.at[pl.ds(0, OUT_CPC)], osem).wait()
        consume(c, c % 2)
    pltpu.make_async_copy(
        bufs[0][3], out.at[pl.ds(0, OUT_CPC)], osem).wait()
    pltpu.make_async_copy(
        bufs[1][3], out.at[pl.ds(0, OUT_CPC)], osem).wait()


_sc_partials = functools.partial(
    pl.kernel,
    out_type=jax.ShapeDtypeStruct((ROWS * LANES // OUT_W, OUT_W), jnp.float32),
    mesh=plsc.VectorSubcoreMesh(core_axis_name="c", subcore_axis_name="s"),
    scratch_types=[
        pltpu.VMEM((RPW,), jnp.int32),
        pltpu.VMEM((RPW,), jnp.int32),
        pltpu.VMEM((RPW,), jnp.int32),
        pltpu.VMEM((CSUB, SUB, HIDDEN), jnp.float32),
        pltpu.VMEM((CSUB, SUB, HIDDEN), jnp.float32),
        pltpu.VMEM((CSUB, SUB, HIDDEN), jnp.float32),
        pltpu.VMEM((CSUB, SUB, HIDDEN), jnp.float32),
        pltpu.VMEM((CSUB, SUB, HIDDEN), jnp.float32),
        pltpu.VMEM((CSUB, SUB, HIDDEN), jnp.float32),
        pltpu.VMEM((OUT_CPC, OUT_W), jnp.float32),
        pltpu.VMEM((OUT_CPC, OUT_W), jnp.float32),
        pltpu.SemaphoreType.DMA,
        pltpu.SemaphoreType.DMA,
        pltpu.SemaphoreType.DMA,
    ],
)(_sc_body)


def _tc_finish(parts_ref, out_ref):
    x = parts_ref[...]
    g = lax.broadcasted_iota(jnp.int32, (OUT_W, OUT_W // LANES), 0)
    h = lax.broadcasted_iota(jnp.int32, (OUT_W, OUT_W // LANES), 1)
    m = (g // LANES == h).astype(jnp.float32)
    s = jax.lax.dot_general(x, m, (((1,), (0,)), ((), ())),
                            preferred_element_type=jnp.float32)
    sc = jnp.sqrt(s)
    half = sc.shape[0] // 2
    val = jnp.maximum(sc[:half] - sc[half:] + MARGIN, 0.0)
    out_ref[0, 0] = jnp.sum(val)


def kernel(pos_h, pos_r, pos_t, neg_h, neg_r, neg_t,
           entity_embeddings, relation_embeddings):
    hidx = jnp.concatenate([pos_h, neg_h]).astype(jnp.int32)
    tidx = jnp.concatenate([pos_t, neg_t]).astype(jnp.int32)
    ridx = jnp.concatenate([pos_r[:, 0], neg_r[:, 0]]).astype(jnp.int32)
    ent3 = entity_embeddings.reshape(ENT_T, SUB, HIDDEN)
    rel3 = relation_embeddings.reshape(REL_T, SUB, HIDDEN)
    parts = _sc_partials(ent3, rel3, hidx, tidx, ridx)
    loss = pl.pallas_call(
        _tc_finish,
        out_shape=jax.ShapeDtypeStruct((1, 1), jnp.float32),
        out_specs=pl.BlockSpec(memory_space=pltpu.SMEM),
    )(parts)
    return loss.reshape(())
